# baseline (speedup 1.0000x reference)
import jax
import jax.numpy as jnp
from jax import lax
from jax.experimental import pallas as pl
from jax.experimental.pallas import tpu as pltpu
from jax.experimental.pallas import tpu_sc as plsc

NC = 2
NS = 16
NW = NC * NS
PAIRS = 16384
D = 32
ROWS = 1000000
RPS = 4
SROWS = ROWS // RPS
PIECE = 128
PAD = 136
SPP = PIECE // RPS
NPIECE = ROWS // PIECE
TAIL = ROWS - NPIECE * PIECE
STAIL = TAIL // RPS
TAIL_W = NPIECE % NW
PAIRS_PER_W = PAIRS // NW
NCHUNK = 8
CHUNK = 2 * PAIRS_PER_W // NCHUNK
CPAIRS = CHUNK // 2
CGROUPS = CPAIRS // 16


def _tp_body(embt_hbm, srow_hbm, in_v, out_v, tin_v, tout_v, sem, osem):
    wid = lax.axis_index("s") * NC + lax.axis_index("c")
    d_lo = lax.iota(jnp.int32, 16)
    d_hi = d_lo + 16
    NPAIR = (NPIECE // NW) // 2
    NEVEN = 2 * NPAIR

    def transpose_static(src, dst, buf, nrows):
        for r in range(nrows):
            rs = jnp.full((16,), r, jnp.int32)
            bs = jnp.full((16,), buf, jnp.int32)
            dst[buf, r // RPS, pl.ds((r % RPS) * D, 16)] = plsc.load_gather(
                src, [bs, d_lo, rs])
            dst[buf, r // RPS, pl.ds((r % RPS) * D + 16, 16)] = (
                plsc.load_gather(src, [bs, d_hi, rs]))

    def issue_piece(p, sub):
        for d in range(D):
            pltpu.async_copy(
                embt_hbm.at[d, pl.ds(p * PIECE, PIECE)],
                in_v.at[sub, d, pl.ds(0, PIECE)], sem,
            )

    def wait_piece(sub):
        pltpu.make_async_copy(
            embt_hbm.at[:, pl.ds(0, PIECE)], out_v.at[sub], sem
        ).wait()

    n_my = NPIECE // NW + jnp.where(
        wid < NPIECE % NW, 1, 0).astype(jnp.int32)

    issue_piece(wid, 0)
    issue_piece(wid + NW, 1)

    def step(i, carry):
        p = wid + i * NW
        buf = lax.rem(i, 2)
        wait_piece(buf)

        @pl.when(i >= 2)
        def _():
            pltpu.make_async_copy(
                out_v.at[buf], srow_hbm.at[pl.ds(0, SPP)], osem
            ).wait()

        transpose_static(in_v, out_v, buf, PIECE)
        pltpu.async_copy(
            out_v.at[buf], srow_hbm.at[pl.ds(p * SPP, SPP)], osem
        )

        @pl.when(i + 2 < n_my)
        def _():
            issue_piece(p + 2 * NW, buf)
        return carry

    lax.fori_loop(0, n_my, step, 0)

    for sub in (0, 1):
        pltpu.make_async_copy(
            out_v.at[sub], srow_hbm.at[pl.ds(0, SPP)], osem
        ).wait()

    @pl.when(wid == TAIL_W)
    def _():
        pltpu.sync_copy(embt_hbm.at[:, pl.ds(NPIECE * PIECE, TAIL)], tin_v)

        def trow(r, carry2):
            rs = jnp.full((16,), r, jnp.int32)
            sr = r // RPS
            cc = (r % RPS) * D
            tout_v[sr, pl.ds(cc, 16)] = plsc.load_gather(tin_v, [d_lo, rs])
            tout_v[sr, pl.ds(cc + 16, 16)] = plsc.load_gather(tin_v, [d_hi, rs])
            return carry2

        lax.fori_loop(0, TAIL, trow, 0, unroll=8)
        pltpu.sync_copy(tout_v, srow_hbm.at[pl.ds(NPIECE * SPP, STAIL)])


def _ip_body(srow_hbm, blk_hbm, sub_hbm, out_hbm, blk_v, sub_v, rows_v, out_v,
             sem):
    wid = lax.axis_index("s") * NC + lax.axis_index("c")
    pltpu.sync_copy(blk_hbm.at[wid], blk_v)
    pltpu.sync_copy(sub_hbm.at[wid], sub_v)

    def issue(j, buf):
        return pltpu.async_copy(
            srow_hbm.at[blk_v.at[j]], rows_v.at[buf], sem
        )

    pending = issue(0, 0)
    for j in range(NCHUNK):
        if j + 1 < NCHUNK:
            nxt = issue(j + 1, (j + 1) % 2)
        pending.wait()
        buf = j % 2

        def group(g, carry):
            base = g * 16
            slot_s = lax.iota(jnp.int32, 16) + base
            slot_o = slot_s + CPAIRS
            col_s = sub_v[j, pl.ds(base, 16)] * D
            col_o = sub_v[j, pl.ds(CPAIRS + base, 16)] * D
            buf_i = jnp.full((16,), buf, jnp.int32)
            acc = jnp.zeros((16,), jnp.float32)
            for d in range(D):
                s_v = plsc.load_gather(rows_v, [buf_i, slot_s, col_s + d])
                o_v = plsc.load_gather(rows_v, [buf_i, slot_o, col_o + d])
                acc = acc + s_v * o_v
            out_v[pl.ds(j * CPAIRS + base, 16)] = 1.0 / (1.0 + jnp.exp(-acc))
            return carry

        lax.fori_loop(0, CGROUPS, group, 0)
        if j + 1 < NCHUNK:
            pending = nxt
    pltpu.sync_copy(out_v, out_hbm.at[pl.ds(wid * PAIRS_PER_W, PAIRS_PER_W)])


@jax.jit
def _ip(emb, batch_ind):
    embt = emb.T
    idx = batch_ind.astype(jnp.int32)
    idx = idx.reshape(NW, NCHUNK, CPAIRS, 2).transpose(0, 1, 3, 2)
    idx = idx.reshape(NW, NCHUNK, CHUNK)
    blk, sub = idx // RPS, idx % RPS

    mesh = plsc.VectorSubcoreMesh(core_axis_name="c", subcore_axis_name="s")
    params = pltpu.CompilerParams(needs_layout_passes=False)
    srow = pl.kernel(
        _tp_body,
        mesh=mesh,
        compiler_params=params,
        out_type=jax.ShapeDtypeStruct((SROWS, 128), jnp.float32),
        scratch_types=[
            pltpu.VMEM((2, D, PAD), jnp.float32),
            pltpu.VMEM((2, SPP, 128), jnp.float32),
            pltpu.VMEM((D, TAIL), jnp.float32),
            pltpu.VMEM((STAIL, 128), jnp.float32),
            pltpu.SemaphoreType.DMA,
            pltpu.SemaphoreType.DMA,
        ],
    )(embt)
    return pl.kernel(
        _ip_body,
        mesh=mesh,
        compiler_params=params,
        out_type=jax.ShapeDtypeStruct((PAIRS,), jnp.float32),
        scratch_types=[
            pltpu.VMEM((NCHUNK, CHUNK), jnp.int32),
            pltpu.VMEM((NCHUNK, CHUNK), jnp.int32),
            pltpu.VMEM((2, CHUNK, 128), jnp.float32),
            pltpu.VMEM((PAIRS_PER_W,), jnp.float32),
            pltpu.SemaphoreType.DMA,
        ],
    )(srow, blk, sub)


def kernel(emb, batch_ind):
    return _ip(emb, batch_ind)


# speedup vs baseline: 1.7328x; 1.7328x over previous
import jax
import jax.numpy as jnp
from jax import lax
from jax.experimental import pallas as pl
from jax.experimental.pallas import tpu as pltpu
from jax.experimental.pallas import tpu_sc as plsc

NC = 2
NS = 16
NW = NC * NS
PAIRS = 16384
D = 32
PAIRS_PER_W = PAIRS // NW
ROWS_PER_W = 2 * PAIRS_PER_W
NCHUNK = 8
CHUNK = ROWS_PER_W // NCHUNK
GROUPS = PAIRS_PER_W // 16


def _ip_body(emb_hbm, idx_hbm, out_hbm, idx_v, rows_v, out_v, sem):
    wid = lax.axis_index("s") * NC + lax.axis_index("c")
    pltpu.sync_copy(idx_hbm.at[wid], idx_v)
    copies = []
    for j in range(NCHUNK):
        copies.append(
            pltpu.async_copy(
                emb_hbm.at[idx_v.at[j]],
                rows_v.at[pl.ds(j * CHUNK, CHUNK)],
                sem,
            )
        )
    for c in copies:
        c.wait()

    even = lax.iota(jnp.int32, 16) * 2

    def group(g, carry):
        row_s = even + g * 32
        row_o = row_s + 1
        acc = jnp.zeros((16,), jnp.float32)
        for d in range(D):
            col = jnp.full((16,), d, jnp.int32)
            s_v = plsc.load_gather(rows_v, [row_s, col])
            o_v = plsc.load_gather(rows_v, [row_o, col])
            acc = acc + s_v * o_v
        out_v[pl.ds(g * 16, 16)] = 1.0 / (1.0 + jnp.exp(-acc))
        return carry

    lax.fori_loop(0, GROUPS, group, 0)
    pltpu.sync_copy(out_v, out_hbm.at[pl.ds(wid * PAIRS_PER_W, PAIRS_PER_W)])


@jax.jit
def _ip(emb, idx):
    mesh = plsc.VectorSubcoreMesh(core_axis_name="c", subcore_axis_name="s")
    return pl.kernel(
        _ip_body,
        mesh=mesh,
        compiler_params=pltpu.CompilerParams(
            needs_layout_passes=False, use_tc_tiling_on_sc=False
        ),
        out_type=jax.ShapeDtypeStruct((PAIRS,), jnp.float32),
        scratch_types=[
            pltpu.VMEM((NCHUNK, CHUNK), jnp.int32),
            pltpu.VMEM((ROWS_PER_W, D), jnp.float32),
            pltpu.VMEM((PAIRS_PER_W,), jnp.float32),
            pltpu.SemaphoreType.DMA,
        ],
    )(emb, idx)


def kernel(emb, batch_ind):
    idx = batch_ind.astype(jnp.int32).reshape(NW, NCHUNK, CHUNK)
    return _ip(emb, idx)
